# Initial kernel scaffold; baseline (speedup 1.0000x reference)
import jax
import jax.numpy as jnp
from jax.experimental import pallas as pl


def kernel(x, batch):
    raise NotImplementedError("write your pallas kernel here")


# speedup vs baseline: 5.3571x; 5.3571x over previous
import functools

import jax
import jax.numpy as jnp
from jax import lax
from jax.experimental import pallas as pl
from jax.experimental.pallas import tpu as pltpu
from jax.experimental.pallas import tpu_sc as plsc

N = 320000
D = 128
S = 10000

NC = 2
NS = 16
NW = NC * NS

B = 128
NCHUNK = N // B
FULL = NCHUNK // NW
REM = NCHUNK % NW
STRIPE = 624


def _zero_rows(ref, nrows, ncols):
    z = jnp.zeros((16,), jnp.float32)
    cb = ncols // 16

    def body(i, carry):
        r = i // cb
        c = i % cb
        ref[r, pl.ds(c * 16, 16)] = z
        return carry

    lax.fori_loop(0, nrows * cb, body, 0)


def _fill_ones(ref, nrows, ncols):
    o = jnp.ones((16,), jnp.float32)
    cb = ncols // 16

    def body(i, carry):
        r = i // cb
        c = i % cb
        ref[r, pl.ds(c * 16, 16)] = o
        return carry

    lax.fori_loop(0, nrows * cb, body, 0)


def _sc_body(x_hbm, ids_hbm, out_s, out_c, sums_sh, cnts_sh, x_v, idx_v,
             ones_v, zrow_v):
    cid = lax.axis_index("c")
    sid = lax.axis_index("s")
    wid = sid * NC + cid

    _zero_rows(x_v, B, D)
    _zero_rows(zrow_v, STRIPE, 16)
    _fill_ones(ones_v, B, 16)
    r0 = sid * STRIPE

    def zbody(k, carry):
        pltpu.sync_copy(x_v.at[pl.ds(0, 16)],
                        sums_sh.at[pl.ds(r0 + k * 16, 16)])
        return carry

    lax.fori_loop(0, STRIPE // 16, zbody, 0)
    pltpu.sync_copy(zrow_v, cnts_sh.at[pl.ds(r0, STRIPE)])

    @pl.when(sid == 0)
    def _():
        pltpu.sync_copy(x_v.at[pl.ds(0, 16)],
                        sums_sh.at[pl.ds(NS * STRIPE, 16)])
        pltpu.sync_copy(zrow_v.at[pl.ds(0, 16)],
                        cnts_sh.at[pl.ds(NS * STRIPE, 16)])

    plsc.subcore_barrier()

    def chunk(ci):
        base = ci * B
        pltpu.sync_copy(ids_hbm.at[pl.ds(base, B)], idx_v)
        pltpu.sync_copy(x_hbm.at[pl.ds(base, B)], x_v)
        pltpu.sync_copy(x_v, sums_sh.at[idx_v], add=True)
        pltpu.sync_copy(ones_v, cnts_sh.at[idx_v], add=True)

    def body(j, carry):
        chunk(wid + NW * j)
        return carry

    lax.fori_loop(0, FULL, body, 0)

    @pl.when(wid < REM)
    def _():
        chunk(NW * FULL + wid)

    plsc.subcore_barrier()
    pltpu.sync_copy(sums_sh.at[pl.ds(r0, STRIPE)],
                    out_s.at[cid, pl.ds(r0, STRIPE)])
    pltpu.sync_copy(cnts_sh.at[pl.ds(r0, STRIPE)],
                    out_c.at[cid, pl.ds(r0, STRIPE)])

    @pl.when(sid == 0)
    def _():
        pltpu.sync_copy(sums_sh.at[pl.ds(NS * STRIPE, 16)],
                        out_s.at[cid, pl.ds(NS * STRIPE, 16)])
        pltpu.sync_copy(cnts_sh.at[pl.ds(NS * STRIPE, 16)],
                        out_c.at[cid, pl.ds(NS * STRIPE, 16)])


_sc_accumulate = pl.kernel(
    _sc_body,
    out_type=(
        jax.ShapeDtypeStruct((NC, S, D), jnp.float32),
        jax.ShapeDtypeStruct((NC, S, 16), jnp.float32),
    ),
    mesh=plsc.VectorSubcoreMesh(core_axis_name="c", subcore_axis_name="s"),
    compiler_params=pltpu.CompilerParams(use_tc_tiling_on_sc=False),
    scratch_types=[
        pltpu.VMEM_SHARED((S, D), jnp.float32),
        pltpu.VMEM_SHARED((S, 16), jnp.float32),
        pltpu.VMEM((B, D), jnp.float32),
        pltpu.VMEM((B,), jnp.int32),
        pltpu.VMEM((B, 16), jnp.float32),
        pltpu.VMEM((STRIPE, 16), jnp.float32),
    ],
)


def _merge_body(s_ref, c_ref, o_ref):
    sums = s_ref[0] + s_ref[1]
    cnt = c_ref[0, :, 0:1] + c_ref[1, :, 0:1]
    o_ref[...] = sums / jnp.maximum(cnt, 1.0)


_merge = pl.pallas_call(
    _merge_body,
    out_shape=jax.ShapeDtypeStruct((S, D), jnp.float32),
)


def kernel(x, batch):
    ids = batch.astype(jnp.int32)
    sums, cnts = _sc_accumulate(x, ids)
    return _merge(sums, cnts)


# speedup vs baseline: 7.2005x; 1.3441x over previous
import jax
import jax.numpy as jnp
from jax import lax
from jax.experimental import pallas as pl
from jax.experimental.pallas import tpu as pltpu
from jax.experimental.pallas import tpu_sc as plsc

N = 320000
D = 128
S = 10000

NC = 2
NS = 16
NW = NC * NS

B = 128
NCHUNK = N // B
FULL = NCHUNK // NW
REM = NCHUNK % NW
NBUF = 2
NGROUP = FULL // NBUF
STRIPE = 624
ZR = 208


def _fill_rows(ref, nrows, ncols, val):
    v = jnp.full((16,), val, jnp.float32)
    cb = ncols // 16

    def body(i, carry):
        r = i // cb
        c = i % cb
        ref[r, pl.ds(c * 16, 16)] = v
        return carry

    lax.fori_loop(0, nrows * cb, body, 0)


def _sc_body(x_hbm, ids_hbm, out_s, out_c, sums_sh, cnts_sh, x_v, idx_v,
             ones_v, zrow_v, gsem, ssem):
    cid = lax.axis_index("c")
    sid = lax.axis_index("s")
    wid = sid * NC + cid
    row0 = wid * (FULL * B)
    chunk0 = wid * FULL

    _fill_rows(x_v.at[0], B, D, 0.0)
    _fill_rows(zrow_v, ZR, 16, 0.0)
    _fill_rows(ones_v, B, 16, 1.0)
    r0 = sid * STRIPE
    for k in range(4):
        pltpu.sync_copy(x_v.at[0], sums_sh.at[pl.ds(r0 + k * B, B)])
    pltpu.sync_copy(x_v.at[0].at[pl.ds(0, STRIPE - 4 * B)],
                    sums_sh.at[pl.ds(r0 + 4 * B, STRIPE - 4 * B)])
    for k in range(3):
        pltpu.sync_copy(zrow_v, cnts_sh.at[pl.ds(r0 + k * ZR, ZR)])

    @pl.when(sid == 0)
    def _():
        pltpu.sync_copy(x_v.at[0].at[pl.ds(0, 16)],
                        sums_sh.at[pl.ds(NS * STRIPE, 16)])
        pltpu.sync_copy(zrow_v.at[pl.ds(0, 16)],
                        cnts_sh.at[pl.ds(NS * STRIPE, 16)])

    plsc.subcore_barrier()

    for b in range(NBUF):
        pltpu.async_copy(x_hbm.at[pl.ds(row0 + b * B, B)], x_v.at[b],
                         gsem.at[b])
        pltpu.async_copy(ids_hbm.at[chunk0 + b], idx_v.at[b], gsem.at[b])

    def group(i, carry):
        for b in range(NBUF):
            j = i * NBUF + b
            pltpu.make_async_copy(x_hbm.at[pl.ds(row0 + j * B, B)],
                                  x_v.at[b], gsem.at[b]).wait()
            pltpu.make_async_copy(ids_hbm.at[chunk0 + j], idx_v.at[b],
                                  gsem.at[b]).wait()
            pltpu.async_copy(x_v.at[b], sums_sh.at[idx_v.at[b]], ssem.at[b],
                             add=True)
            pltpu.async_copy(ones_v, cnts_sh.at[idx_v.at[b]], ssem.at[b],
                             add=True)
        for b in range(NBUF):
            j2 = (i + 1) * NBUF + b
            pltpu.make_async_copy(x_v.at[b], sums_sh.at[idx_v.at[b]],
                                  ssem.at[b]).wait()
            pltpu.make_async_copy(ones_v, cnts_sh.at[idx_v.at[b]],
                                  ssem.at[b]).wait()

            @pl.when(j2 < FULL)
            def _():
                pltpu.async_copy(x_hbm.at[pl.ds(row0 + j2 * B, B)],
                                 x_v.at[b], gsem.at[b])
                pltpu.async_copy(ids_hbm.at[chunk0 + j2], idx_v.at[b],
                                 gsem.at[b])
        return carry

    lax.fori_loop(0, NGROUP, group, 0)

    @pl.when(wid < REM)
    def _():
        base = (NW * FULL + wid) * B
        pltpu.sync_copy(ids_hbm.at[NW * FULL + wid], idx_v.at[0])
        pltpu.sync_copy(x_hbm.at[pl.ds(base, B)], x_v.at[0])
        pltpu.sync_copy(x_v.at[0], sums_sh.at[idx_v.at[0]], add=True)
        pltpu.sync_copy(ones_v, cnts_sh.at[idx_v.at[0]], add=True)

    plsc.subcore_barrier()
    pltpu.sync_copy(sums_sh.at[pl.ds(r0, STRIPE)],
                    out_s.at[cid, pl.ds(r0, STRIPE)])
    pltpu.sync_copy(cnts_sh.at[pl.ds(r0, STRIPE)],
                    out_c.at[cid, pl.ds(r0, STRIPE)])

    @pl.when(sid == 0)
    def _():
        pltpu.sync_copy(sums_sh.at[pl.ds(NS * STRIPE, 16)],
                        out_s.at[cid, pl.ds(NS * STRIPE, 16)])
        pltpu.sync_copy(cnts_sh.at[pl.ds(NS * STRIPE, 16)],
                        out_c.at[cid, pl.ds(NS * STRIPE, 16)])


_sc_accumulate = pl.kernel(
    _sc_body,
    out_type=(
        jax.ShapeDtypeStruct((NC, S, D), jnp.float32),
        jax.ShapeDtypeStruct((NC, S, 16), jnp.float32),
    ),
    mesh=plsc.VectorSubcoreMesh(core_axis_name="c", subcore_axis_name="s"),
    compiler_params=pltpu.CompilerParams(use_tc_tiling_on_sc=False),
    scratch_types=[
        pltpu.VMEM_SHARED((S, D), jnp.float32),
        pltpu.VMEM_SHARED((S, 16), jnp.float32),
        pltpu.VMEM((NBUF, B, D), jnp.float32),
        pltpu.VMEM((NBUF, B), jnp.int32),
        pltpu.VMEM((B, 16), jnp.float32),
        pltpu.VMEM((ZR, 16), jnp.float32),
        pltpu.SemaphoreType.DMA((NBUF,)),
        pltpu.SemaphoreType.DMA((NBUF,)),
    ],
)


def _merge_body(s_ref, c_ref, o_ref):
    sums = s_ref[0] + s_ref[1]
    cnt = c_ref[0, :, 0:1] + c_ref[1, :, 0:1]
    o_ref[...] = sums / jnp.maximum(cnt, 1.0)


_merge = pl.pallas_call(
    _merge_body,
    out_shape=jax.ShapeDtypeStruct((S, D), jnp.float32),
)


def kernel(x, batch):
    ids = batch.astype(jnp.int32).reshape(NCHUNK, B)
    sums, cnts = _sc_accumulate(x, ids)
    return _merge(sums, cnts)


# speedup vs baseline: 8.5375x; 1.1857x over previous
import jax
import jax.numpy as jnp
from jax import lax
from jax.experimental import pallas as pl
from jax.experimental.pallas import tpu as pltpu
from jax.experimental.pallas import tpu_sc as plsc

N = 320000
D = 128
S = 10000

NC = 2
NS = 16
NW = NC * NS

B = 80
NCHUNK = N // B
FULL = NCHUNK // NW
NBUF = 3
NGROUP = 41
STRIPE = 624
ZR = 208


def _fill_rows(ref, nrows, ncols, val):
    v = jnp.full((16,), val, jnp.float32)
    cb = ncols // 16

    def body(i, carry):
        r = i // cb
        c = i % cb
        ref[r, pl.ds(c * 16, 16)] = v
        return carry

    lax.fori_loop(0, nrows * cb, body, 0)


def _sc_body(x_hbm, ids_hbm, out_s, out_c, sums_sh, cnts_sh, x_v, idx_v,
             ones_v, zrow_v, gsem, ssem):
    cid = lax.axis_index("c")
    sid = lax.axis_index("s")
    wid = sid * NC + cid
    row0 = wid * (FULL * B)
    chunk0 = wid * FULL

    _fill_rows(x_v.at[0], B, D, 0.0)
    _fill_rows(zrow_v, ZR, 16, 0.0)
    _fill_rows(ones_v, B, 16, 1.0)
    r0 = sid * STRIPE
    for k in range(7):
        pltpu.sync_copy(x_v.at[0], sums_sh.at[pl.ds(r0 + k * B, B)])
    pltpu.sync_copy(x_v.at[0].at[pl.ds(0, STRIPE - 7 * B)],
                    sums_sh.at[pl.ds(r0 + 7 * B, STRIPE - 7 * B)])
    for k in range(3):
        pltpu.sync_copy(zrow_v, cnts_sh.at[pl.ds(r0 + k * ZR, ZR)])

    @pl.when(sid == 0)
    def _():
        pltpu.sync_copy(x_v.at[0].at[pl.ds(0, 16)],
                        sums_sh.at[pl.ds(NS * STRIPE, 16)])
        pltpu.sync_copy(zrow_v.at[pl.ds(0, 16)],
                        cnts_sh.at[pl.ds(NS * STRIPE, 16)])

    plsc.subcore_barrier()

    for b in range(NBUF):
        pltpu.async_copy(x_hbm.at[pl.ds(row0 + b * B, B)], x_v.at[b],
                         gsem.at[b])
        pltpu.async_copy(ids_hbm.at[chunk0 + b], idx_v.at[b], gsem.at[b])

    def group(i, carry):
        for b in range(NBUF):
            j = i * NBUF + b
            pltpu.make_async_copy(x_hbm.at[pl.ds(row0 + j * B, B)],
                                  x_v.at[b], gsem.at[b]).wait()
            pltpu.make_async_copy(ids_hbm.at[chunk0 + j], idx_v.at[b],
                                  gsem.at[b]).wait()
            pltpu.async_copy(x_v.at[b], sums_sh.at[idx_v.at[b]], ssem.at[b],
                             add=True)
            pltpu.async_copy(ones_v, cnts_sh.at[idx_v.at[b]], ssem.at[b],
                             add=True)
        for b in range(NBUF):
            j2 = (i + 1) * NBUF + b
            pltpu.make_async_copy(x_v.at[b], sums_sh.at[idx_v.at[b]],
                                  ssem.at[b]).wait()
            pltpu.make_async_copy(ones_v, cnts_sh.at[idx_v.at[b]],
                                  ssem.at[b]).wait()

            @pl.when(j2 < FULL)
            def _():
                pltpu.async_copy(x_hbm.at[pl.ds(row0 + j2 * B, B)],
                                 x_v.at[b], gsem.at[b])
                pltpu.async_copy(ids_hbm.at[chunk0 + j2], idx_v.at[b],
                                 gsem.at[b])
        return carry

    lax.fori_loop(0, NGROUP, group, 0)

    for b in range(FULL - NGROUP * NBUF):
        j = NGROUP * NBUF + b
        pltpu.make_async_copy(x_hbm.at[pl.ds(row0 + j * B, B)],
                              x_v.at[b], gsem.at[b]).wait()
        pltpu.make_async_copy(ids_hbm.at[chunk0 + j], idx_v.at[b],
                              gsem.at[b]).wait()
        pltpu.sync_copy(x_v.at[b], sums_sh.at[idx_v.at[b]], add=True)
        pltpu.sync_copy(ones_v, cnts_sh.at[idx_v.at[b]], add=True)

    plsc.subcore_barrier()
    pltpu.sync_copy(sums_sh.at[pl.ds(r0, STRIPE)],
                    out_s.at[cid, pl.ds(r0, STRIPE)])
    pltpu.sync_copy(cnts_sh.at[pl.ds(r0, STRIPE)],
                    out_c.at[cid, pl.ds(r0, STRIPE)])

    @pl.when(sid == 0)
    def _():
        pltpu.sync_copy(sums_sh.at[pl.ds(NS * STRIPE, 16)],
                        out_s.at[cid, pl.ds(NS * STRIPE, 16)])
        pltpu.sync_copy(cnts_sh.at[pl.ds(NS * STRIPE, 16)],
                        out_c.at[cid, pl.ds(NS * STRIPE, 16)])


_sc_accumulate = pl.kernel(
    _sc_body,
    out_type=(
        jax.ShapeDtypeStruct((NC, S, D), jnp.float32),
        jax.ShapeDtypeStruct((NC, S, 16), jnp.float32),
    ),
    mesh=plsc.VectorSubcoreMesh(core_axis_name="c", subcore_axis_name="s"),
    compiler_params=pltpu.CompilerParams(use_tc_tiling_on_sc=False),
    scratch_types=[
        pltpu.VMEM_SHARED((S, D), jnp.float32),
        pltpu.VMEM_SHARED((S, 16), jnp.float32),
        pltpu.VMEM((NBUF, B, D), jnp.float32),
        pltpu.VMEM((NBUF, B), jnp.int32),
        pltpu.VMEM((B, 16), jnp.float32),
        pltpu.VMEM((ZR, 16), jnp.float32),
        pltpu.SemaphoreType.DMA((NBUF,)),
        pltpu.SemaphoreType.DMA((NBUF,)),
    ],
)


def _merge_body(s_ref, c_ref, o_ref):
    sums = s_ref[0] + s_ref[1]
    cnt = c_ref[0, :, 0:1] + c_ref[1, :, 0:1]
    o_ref[...] = sums / jnp.maximum(cnt, 1.0)


_merge = pl.pallas_call(
    _merge_body,
    out_shape=jax.ShapeDtypeStruct((S, D), jnp.float32),
)


def kernel(x, batch):
    ids = batch.astype(jnp.int32).reshape(NCHUNK, B)
    sums, cnts = _sc_accumulate(x, ids)
    return _merge(sums, cnts)


# speedup vs baseline: 8.6110x; 1.0086x over previous
import jax
import jax.numpy as jnp
from jax import lax
from jax.experimental import pallas as pl
from jax.experimental.pallas import tpu as pltpu
from jax.experimental.pallas import tpu_sc as plsc

N = 320000
D = 128
S = 10000

NC = 2
NS = 16
NW = NC * NS

B = 80
NCHUNK = N // B
FULL = NCHUNK // NW
NBUF = 3
NGROUP = 41
STRIPE = 624
ZR = 208


def _fill_rows(ref, nrows, ncols, val):
    v = jnp.full((16,), val, jnp.float32)
    cb = ncols // 16

    def body(i, carry):
        r = i // cb
        c = i % cb
        ref[r, pl.ds(c * 16, 16)] = v
        return carry

    lax.fori_loop(0, nrows * cb, body, 0)


def _sc_body(x_hbm, ids_hbm, out_s, out_c, sums_sh, cnts_sh, x_v, idx_v,
             ones_v, zrow_v, gsem, ssem):
    cid = lax.axis_index("c")
    sid = lax.axis_index("s")
    wid = sid * NC + cid
    row0 = wid * (FULL * B)
    chunk0 = wid * FULL

    _fill_rows(x_v.at[0], B, D, 0.0)
    _fill_rows(zrow_v, ZR, 16, 0.0)
    _fill_rows(ones_v, B, 16, 1.0)
    r0 = sid * STRIPE
    zcopies = []
    for k in range(7):
        zcopies.append((x_v.at[0], sums_sh.at[pl.ds(r0 + k * B, B)]))
    zcopies.append((x_v.at[0].at[pl.ds(0, STRIPE - 7 * B)],
                    sums_sh.at[pl.ds(r0 + 7 * B, STRIPE - 7 * B)]))
    for k in range(3):
        zcopies.append((zrow_v, cnts_sh.at[pl.ds(r0 + k * ZR, ZR)]))
    for src, dst in zcopies:
        pltpu.async_copy(src, dst, ssem.at[0])
    for src, dst in zcopies:
        pltpu.make_async_copy(src, dst, ssem.at[0]).wait()

    @pl.when(sid == 0)
    def _():
        pltpu.sync_copy(x_v.at[0].at[pl.ds(0, 16)],
                        sums_sh.at[pl.ds(NS * STRIPE, 16)])
        pltpu.sync_copy(zrow_v.at[pl.ds(0, 16)],
                        cnts_sh.at[pl.ds(NS * STRIPE, 16)])

    plsc.subcore_barrier()

    for b in range(NBUF):
        pltpu.async_copy(x_hbm.at[pl.ds(row0 + b * B, B)], x_v.at[b],
                         gsem.at[b])
        pltpu.async_copy(ids_hbm.at[chunk0 + b], idx_v.at[b], gsem.at[b])

    def group(i, carry):
        for b in range(NBUF):
            j = i * NBUF + b
            pltpu.make_async_copy(x_hbm.at[pl.ds(row0 + j * B, B)],
                                  x_v.at[b], gsem.at[b]).wait()
            pltpu.make_async_copy(ids_hbm.at[chunk0 + j], idx_v.at[b],
                                  gsem.at[b]).wait()
            pltpu.async_copy(x_v.at[b], sums_sh.at[idx_v.at[b]], ssem.at[b],
                             add=True)
            pltpu.async_copy(ones_v, cnts_sh.at[idx_v.at[b]], ssem.at[b],
                             add=True)
        for b in range(NBUF):
            j2 = (i + 1) * NBUF + b
            pltpu.make_async_copy(x_v.at[b], sums_sh.at[idx_v.at[b]],
                                  ssem.at[b]).wait()
            pltpu.make_async_copy(ones_v, cnts_sh.at[idx_v.at[b]],
                                  ssem.at[b]).wait()

            @pl.when(j2 < FULL)
            def _():
                pltpu.async_copy(x_hbm.at[pl.ds(row0 + j2 * B, B)],
                                 x_v.at[b], gsem.at[b])
                pltpu.async_copy(ids_hbm.at[chunk0 + j2], idx_v.at[b],
                                 gsem.at[b])
        return carry

    lax.fori_loop(0, NGROUP, group, 0)

    for b in range(FULL - NGROUP * NBUF):
        j = NGROUP * NBUF + b
        pltpu.make_async_copy(x_hbm.at[pl.ds(row0 + j * B, B)],
                              x_v.at[b], gsem.at[b]).wait()
        pltpu.make_async_copy(ids_hbm.at[chunk0 + j], idx_v.at[b],
                              gsem.at[b]).wait()
        pltpu.sync_copy(x_v.at[b], sums_sh.at[idx_v.at[b]], add=True)
        pltpu.sync_copy(ones_v, cnts_sh.at[idx_v.at[b]], add=True)

    plsc.subcore_barrier()
    wcopies = [
        (sums_sh.at[pl.ds(r0, STRIPE)], out_s.at[cid, pl.ds(r0, STRIPE)]),
        (cnts_sh.at[pl.ds(r0, STRIPE)], out_c.at[cid, pl.ds(r0, STRIPE)]),
    ]
    for src, dst in wcopies:
        pltpu.async_copy(src, dst, gsem.at[0])

    @pl.when(sid == 0)
    def _():
        pltpu.sync_copy(sums_sh.at[pl.ds(NS * STRIPE, 16)],
                        out_s.at[cid, pl.ds(NS * STRIPE, 16)])
        pltpu.sync_copy(cnts_sh.at[pl.ds(NS * STRIPE, 16)],
                        out_c.at[cid, pl.ds(NS * STRIPE, 16)])

    for src, dst in wcopies:
        pltpu.make_async_copy(src, dst, gsem.at[0]).wait()


_sc_accumulate = pl.kernel(
    _sc_body,
    out_type=(
        jax.ShapeDtypeStruct((NC, S, D), jnp.float32),
        jax.ShapeDtypeStruct((NC, S, 16), jnp.float32),
    ),
    mesh=plsc.VectorSubcoreMesh(core_axis_name="c", subcore_axis_name="s"),
    compiler_params=pltpu.CompilerParams(use_tc_tiling_on_sc=False),
    scratch_types=[
        pltpu.VMEM_SHARED((S, D), jnp.float32),
        pltpu.VMEM_SHARED((S, 16), jnp.float32),
        pltpu.VMEM((NBUF, B, D), jnp.float32),
        pltpu.VMEM((NBUF, B), jnp.int32),
        pltpu.VMEM((B, 16), jnp.float32),
        pltpu.VMEM((ZR, 16), jnp.float32),
        pltpu.SemaphoreType.DMA((NBUF,)),
        pltpu.SemaphoreType.DMA((NBUF,)),
    ],
)


def _merge_body(s_ref, c_ref, o_ref):
    sums = s_ref[0] + s_ref[1]
    cnt = c_ref[0, :, 0:1] + c_ref[1, :, 0:1]
    o_ref[...] = sums / jnp.maximum(cnt, 1.0)


_merge = pl.pallas_call(
    _merge_body,
    out_shape=jax.ShapeDtypeStruct((S, D), jnp.float32),
)


def kernel(x, batch):
    ids = batch.astype(jnp.int32).reshape(NCHUNK, B)
    sums, cnts = _sc_accumulate(x, ids)
    return _merge(sums, cnts)
